# Initial kernel scaffold; baseline (speedup 1.0000x reference)
import jax
import jax.numpy as jnp
from jax.experimental import pallas as pl


def kernel(x, edge_index, batch, W0, b0, g0, be0, W1, b1, g1, be1, W2, b2, g2, be2, fW1, fb1, fW2, fb2, fWo, fbo):
    raise NotImplementedError("write your pallas kernel here")


# speedup vs baseline: 21.7446x; 21.7446x over previous
import functools

import jax
import jax.numpy as jnp
from jax import lax
from jax.experimental import pallas as pl
from jax.experimental.pallas import tpu as pltpu
from jax.experimental.pallas import tpu_sc as plsc

N = 10000
E = 320000
D = 128
HD = D // 2
G = 64
G1 = G + 1
NC = 2
NS = 16
L = 16
NW = NC * NS
NP = 10240
RPT = NP // NW
CHUNK = 128
KCH = 158
EP = NS * KCH * CHUNK
KD = KCH // 2

_mesh = plsc.VectorSubcoreMesh(core_axis_name="c", subcore_axis_name="s")

_f32 = jnp.float32
_i32 = jnp.int32


@functools.partial(
    pl.kernel,
    out_type=jax.ShapeDtypeStruct((NC, NP), _f32),
    mesh=_mesh,
    scratch_types=[
        pltpu.VMEM((KD, CHUNK), _i32),
        pltpu.VMEM((CHUNK,), _f32),
        pltpu.VMEM((NP // NS,), _f32),
        pltpu.VMEM_SHARED((NP,), _f32),
    ],
)
def _deg_sc(dsts_hbm, out_hbm, idx_v, ones_v, buf_v, acc_sh):
    c = lax.axis_index("c")
    s = lax.axis_index("s")
    for k in range(CHUNK // L):
        ones_v[pl.ds(k * L, L)] = jnp.ones((L,), _f32)

    def _z(i, carry):
        buf_v[pl.ds(i * L, L)] = jnp.zeros((L,), _f32)
        return carry

    lax.fori_loop(0, (NP // NS) // L, _z, 0)
    pltpu.sync_copy(buf_v, acc_sh.at[pl.ds(s * (NP // NS), NP // NS)])
    w = c * NS + s
    pltpu.sync_copy(dsts_hbm.at[w], idx_v)
    plsc.subcore_barrier()
    for j in range(KD):
        pltpu.sync_copy(ones_v, acc_sh.at[idx_v.at[j]], add=True)
    plsc.subcore_barrier()
    pltpu.sync_copy(acc_sh.at[pl.ds(s * (NP // NS), NP // NS)], buf_v)
    pltpu.sync_copy(buf_v, out_hbm.at[c, pl.ds(s * (NP // NS), NP // NS)])


@functools.partial(
    pl.kernel,
    out_type=jax.ShapeDtypeStruct((NC, NP, HD), _f32),
    mesh=_mesh,
    scratch_types=[
        pltpu.VMEM((KCH, CHUNK), _i32),
        pltpu.VMEM((KCH, CHUNK), _i32),
        pltpu.VMEM((2, CHUNK, HD), _f32),
        pltpu.VMEM((L, HD), _f32),
        pltpu.VMEM((CHUNK, HD), _f32),
        pltpu.VMEM_SHARED((NP, HD), _f32),
        pltpu.SemaphoreType.DMA,
        pltpu.SemaphoreType.DMA,
    ],
    compiler_params=pltpu.CompilerParams(use_tc_tiling_on_sc=False),
)
def _agg_sc(hn_hbm, srcs_hbm, dsts_hbm, out_hbm,
            si_v, di_v, rows_v, zb_v, wb_v, acc_sh, sem0, sem1):
    c = lax.axis_index("c")
    s = lax.axis_index("s")
    rows_per_tile = NP // NS
    for k in range(L):
        for j in range(HD // L):
            zb_v[k, pl.ds(j * L, L)] = jnp.zeros((L,), _f32)
    for k in range(rows_per_tile // L):
        pltpu.sync_copy(zb_v, acc_sh.at[pl.ds(s * rows_per_tile + k * L, L)])
    pltpu.sync_copy(srcs_hbm.at[s], si_v)
    pltpu.sync_copy(dsts_hbm.at[s], di_v)
    plsc.subcore_barrier()
    sems = (sem0, sem1)
    cps = [None, None]
    cps[0] = pltpu.async_copy(hn_hbm.at[c].at[si_v.at[0]], rows_v.at[0], sem0)
    for j in range(KCH):
        b = j % 2
        if j + 1 < KCH:
            nb = (j + 1) % 2
            cps[nb] = pltpu.async_copy(hn_hbm.at[c].at[si_v.at[j + 1]],
                                       rows_v.at[nb], sems[nb])
        cps[b].wait()
        pltpu.sync_copy(rows_v.at[b], acc_sh.at[di_v.at[j]], add=True)
    plsc.subcore_barrier()
    for k in range(rows_per_tile // CHUNK):
        r0 = s * rows_per_tile + k * CHUNK
        pltpu.sync_copy(acc_sh.at[pl.ds(r0, CHUNK)], wb_v)
        pltpu.sync_copy(wb_v, out_hbm.at[c, pl.ds(r0, CHUNK)])


@functools.partial(
    pl.kernel,
    out_type=(
        jax.ShapeDtypeStruct((NW, G1 * D), _f32),
        jax.ShapeDtypeStruct((NW, G1 * D), _f32),
        jax.ShapeDtypeStruct((NW, G1 * D), _f32),
    ),
    mesh=_mesh,
    scratch_types=[
        pltpu.VMEM((RPT * D,), _f32),
        pltpu.VMEM((RPT,), _i32),
        pltpu.VMEM((G1 * D,), _f32),
        pltpu.VMEM((G1 * D,), _f32),
        pltpu.VMEM((G1 * D,), _f32),
    ],
    compiler_params=pltpu.CompilerParams(needs_layout_passes=False),
)
def _pool_sc(h_hbm, batch_hbm, osum, omax, ocnt, rows_v, b_v, ls, lm, lc):
    c = lax.axis_index("c")
    s = lax.axis_index("s")
    w = c * NS + s
    pltpu.sync_copy(h_hbm.at[pl.ds(w * RPT * D, RPT * D)], rows_v)
    pltpu.sync_copy(batch_hbm.at[pl.ds(w * RPT, RPT)], b_v)

    def _zi(i, carry):
        ls[pl.ds(i * L, L)] = jnp.zeros((L,), _f32)
        lm[pl.ds(i * L, L)] = jnp.full((L,), -jnp.inf, _f32)
        lc[pl.ds(i * L, L)] = jnp.zeros((L,), _f32)
        return carry

    lax.fori_loop(0, (G1 * D) // L, _zi, 0)

    iota = lax.iota(_i32, L)
    ones = jnp.ones((L,), _f32)

    def _grp(g, carry):
        bvec = b_v[pl.ds(g * L, L)]
        for r in range(L):
            seg_b = jnp.take_along_axis(bvec, jnp.full((L,), r, _i32),
                                        axis=0, mode="promise_in_bounds")
            base = (g * L + r) * D
            sidx0 = seg_b * D + iota
            for j in range(D // L):
                rv = rows_v[pl.ds(base + j * L, L)]
                idx = sidx0 + (j * L)
                plsc.addupdate_scatter(ls, [idx], rv)
                curm = plsc.load_gather(lm, [idx])
                plsc.store_scatter(lm, [idx], jnp.maximum(curm, rv))
            plsc.addupdate_scatter(lc, [sidx0], ones)
        return carry

    lax.fori_loop(0, RPT // L, _grp, 0)
    pltpu.sync_copy(ls, osum.at[w])
    pltpu.sync_copy(lm, omax.at[w])
    pltpu.sync_copy(lc, ocnt.at[w])


def _stage0_body(degp, xp, w0, dinv_o, hn_o):
    deg = (degp[0, :] + degp[1, :] + 1.0).reshape(NP, 1)
    row = lax.broadcasted_iota(_i32, (NP, 1), 0)
    dinv = jnp.where(row < N, lax.rsqrt(deg), 0.0)
    dinv_o[...] = dinv
    hn = jnp.dot(xp[...], w0[...], preferred_element_type=_f32) * dinv
    hn_o[0] = hn[:, :HD]
    hn_o[1] = hn[:, HD:]


def _stage0(degp, xp, w0):
    return pl.pallas_call(
        _stage0_body,
        out_shape=(
            jax.ShapeDtypeStruct((NP, 1), _f32),
            jax.ShapeDtypeStruct((NC, NP, HD), _f32),
        ),
    )(degp, xp, w0)


def _stage_mid_body(accp, hn2, dinv, b, gm, be, wn, hn_o):
    row = lax.broadcasted_iota(_i32, (NP, 1), 0)
    acc = jnp.concatenate([accp[0], accp[1]], axis=1)
    hn = jnp.concatenate([hn2[0], hn2[1]], axis=1)
    pre = (acc + hn) * dinv[...] + b[...].reshape(1, D)
    pre = jnp.where(row < N, pre, 0.0)
    m = jnp.sum(pre, axis=0, keepdims=True) / N
    v = jnp.sum(pre * pre, axis=0, keepdims=True) / N - m * m
    y = (pre - m) * lax.rsqrt(v + 1e-5) * gm[...].reshape(1, D) + be[...].reshape(1, D)
    y = jnp.where(row < N, jnp.maximum(y, 0.0), 0.0)
    hn_n = jnp.dot(y, wn[...], preferred_element_type=_f32) * dinv[...]
    hn_o[0] = hn_n[:, :HD]
    hn_o[1] = hn_n[:, HD:]


def _stage_mid(accp, hn2, dinv, b, gm, be, wn):
    return pl.pallas_call(
        _stage_mid_body,
        out_shape=jax.ShapeDtypeStruct((NC, NP, HD), _f32),
    )(accp, hn2, dinv, b, gm, be, wn)


def _stage_last_body(accp, hn2, dinv, b, gm, be, h_o):
    row = lax.broadcasted_iota(_i32, (NP, 1), 0)
    acc = jnp.concatenate([accp[0], accp[1]], axis=1)
    hn = jnp.concatenate([hn2[0], hn2[1]], axis=1)
    pre = (acc + hn) * dinv[...] + b[...].reshape(1, D)
    pre = jnp.where(row < N, pre, 0.0)
    m = jnp.sum(pre, axis=0, keepdims=True) / N
    v = jnp.sum(pre * pre, axis=0, keepdims=True) / N - m * m
    y = (pre - m) * lax.rsqrt(v + 1e-5) * gm[...].reshape(1, D) + be[...].reshape(1, D)
    h_o[...] = jnp.where(row < N, jnp.maximum(y, 0.0), 0.0)


def _stage_last(accp, hn2, dinv, b, gm, be):
    return pl.pallas_call(
        _stage_last_body,
        out_shape=jax.ShapeDtypeStruct((NP, D), _f32),
    )(accp, hn2, dinv, b, gm, be)


def _head_body(psum, pmax, pcnt, fw1, fb1, fw2, fb2, fwo, fbo, out_o):
    ssum = jnp.sum(psum[...], axis=0).reshape(G1, D)[:G]
    smax = jnp.max(pmax[...], axis=0).reshape(G1, D)[:G]
    scnt = jnp.max(jnp.sum(pcnt[...], axis=0).reshape(G1, D),
                   axis=1, keepdims=True)[:G]
    mean = ssum / jnp.clip(scnt, 1.0)
    z = jnp.concatenate([mean, smax], axis=1)
    z = jnp.maximum(jnp.dot(z, fw1[...], preferred_element_type=_f32)
                    + fb1[...].reshape(1, D), 0.0)
    z = jnp.maximum(jnp.dot(z, fw2[...], preferred_element_type=_f32)
                    + fb2[...].reshape(1, D // 2), 0.0)
    out_o[...] = (jnp.dot(z, fwo[...], preferred_element_type=_f32)
                  + fbo[...].reshape(1, 5))


def _head(psum, pmax, pcnt, fw1, fb1, fw2, fb2, fwo, fbo):
    return pl.pallas_call(
        _head_body,
        out_shape=jax.ShapeDtypeStruct((G, 5), _f32),
    )(psum, pmax, pcnt, fw1, fb1, fw2, fb2, fwo, fbo)


def kernel(x, edge_index, batch, W0, b0, g0, be0, W1, b1, g1, be1,
           W2, b2, g2, be2, fW1, fb1, fW2, fb2, fWo, fbo):
    src = edge_index[0]
    dst = edge_index[1]
    npad = EP - E
    fill = jnp.arange(npad, dtype=_i32)
    src_p = jnp.concatenate([src, fill % N]).reshape(NS, KCH, CHUNK)
    dst_p = jnp.concatenate([dst, N + fill % (NP - N)]).reshape(NS, KCH, CHUNK)
    xp = jnp.pad(x, ((0, NP - N), (0, 0)))
    batch_p = jnp.concatenate([batch, jnp.full((NP - N,), G, _i32)])

    degp = _deg_sc(dst_p.reshape(NW, KD, CHUNK))
    dinv, hn2 = _stage0(degp, xp, W0)

    accp = _agg_sc(hn2, src_p, dst_p)
    hn2 = _stage_mid(accp, hn2, dinv, b0, g0, be0, W1)
    accp = _agg_sc(hn2, src_p, dst_p)
    hn2 = _stage_mid(accp, hn2, dinv, b1, g1, be1, W2)
    accp = _agg_sc(hn2, src_p, dst_p)
    h3 = _stage_last(accp, hn2, dinv, b2, g2, be2)

    psum, pmax, pcnt = _pool_sc(h3.reshape(NP * D), batch_p)
    return _head(psum, pmax, pcnt, fW1, fb1, fW2, fb2, fWo, fbo)


# speedup vs baseline: 24.5637x; 1.1296x over previous
import functools

import jax
import jax.numpy as jnp
from jax import lax
from jax.experimental import pallas as pl
from jax.experimental.pallas import tpu as pltpu
from jax.experimental.pallas import tpu_sc as plsc

N = 10000
E = 320000
D = 128
HD = D // 2
G = 64
G1 = G + 1
NC = 2
NS = 16
L = 16
NW = NC * NS
NP = 10240
RPT = NP // NW
CHUNK = 128
KCH = 158
EP = NS * KCH * CHUNK
KD = KCH // 2

_mesh = plsc.VectorSubcoreMesh(core_axis_name="c", subcore_axis_name="s")

_f32 = jnp.float32
_i32 = jnp.int32


@functools.partial(
    pl.kernel,
    out_type=jax.ShapeDtypeStruct((NC, NP), _f32),
    mesh=_mesh,
    scratch_types=[
        pltpu.VMEM((KD, CHUNK), _i32),
        pltpu.VMEM((CHUNK,), _f32),
        pltpu.VMEM((NP // NS,), _f32),
        pltpu.VMEM_SHARED((NP,), _f32),
    ],
)
def _deg_sc(dsts_hbm, out_hbm, idx_v, ones_v, buf_v, acc_sh):
    c = lax.axis_index("c")
    s = lax.axis_index("s")
    for k in range(CHUNK // L):
        ones_v[pl.ds(k * L, L)] = jnp.ones((L,), _f32)

    def _z(i, carry):
        buf_v[pl.ds(i * L, L)] = jnp.zeros((L,), _f32)
        return carry

    lax.fori_loop(0, (NP // NS) // L, _z, 0)
    pltpu.sync_copy(buf_v, acc_sh.at[pl.ds(s * (NP // NS), NP // NS)])
    w = c * NS + s
    pltpu.sync_copy(dsts_hbm.at[w], idx_v)
    plsc.subcore_barrier()
    for j in range(KD):
        pltpu.sync_copy(ones_v, acc_sh.at[idx_v.at[j]], add=True)
    plsc.subcore_barrier()
    pltpu.sync_copy(acc_sh.at[pl.ds(s * (NP // NS), NP // NS)], buf_v)
    pltpu.sync_copy(buf_v, out_hbm.at[c, pl.ds(s * (NP // NS), NP // NS)])


@functools.partial(
    pl.kernel,
    out_type=jax.ShapeDtypeStruct((NC, NP, HD), _f32),
    mesh=_mesh,
    scratch_types=[
        pltpu.VMEM((KCH, CHUNK), _i32),
        pltpu.VMEM((KCH, CHUNK), _i32),
        pltpu.VMEM((4, CHUNK, HD), _f32),
        pltpu.VMEM((L, HD), _f32),
        pltpu.VMEM((CHUNK, HD), _f32),
        pltpu.VMEM_SHARED((NP, HD), _f32),
    ] + [pltpu.SemaphoreType.DMA] * 8,
    compiler_params=pltpu.CompilerParams(use_tc_tiling_on_sc=False),
)
def _agg_sc(hn_hbm, srcs_hbm, dsts_hbm, out_hbm,
            si_v, di_v, rows_v, zb_v, wb_v, acc_sh, *sems):
    c = lax.axis_index("c")
    s = lax.axis_index("s")
    rows_per_tile = NP // NS
    for k in range(L):
        for j in range(HD // L):
            zb_v[k, pl.ds(j * L, L)] = jnp.zeros((L,), _f32)
    for k in range(rows_per_tile // L):
        pltpu.sync_copy(zb_v, acc_sh.at[pl.ds(s * rows_per_tile + k * L, L)])
    pltpu.sync_copy(srcs_hbm.at[s], si_v)
    pltpu.sync_copy(dsts_hbm.at[s], di_v)
    plsc.subcore_barrier()
    NB = 4
    LAG = 2
    gsem = sems[:NB]
    ssem = sems[NB:]
    gcp = [None] * NB
    scp = [None] * NB
    for i in range(KCH + LAG):
        b = i % NB
        if i < KCH:
            if i >= NB:
                scp[b].wait()
            gcp[b] = pltpu.async_copy(hn_hbm.at[c].at[si_v.at[i]],
                                      rows_v.at[b], gsem[b])
        if i >= LAG:
            j = i - LAG
            bj = j % NB
            gcp[bj].wait()
            scp[bj] = pltpu.async_copy(rows_v.at[bj], acc_sh.at[di_v.at[j]],
                                       ssem[bj], add=True)
    for j in range(KCH - NB, KCH):
        scp[j % NB].wait()
    plsc.subcore_barrier()
    for k in range(rows_per_tile // CHUNK):
        r0 = s * rows_per_tile + k * CHUNK
        pltpu.sync_copy(acc_sh.at[pl.ds(r0, CHUNK)], wb_v)
        pltpu.sync_copy(wb_v, out_hbm.at[c, pl.ds(r0, CHUNK)])


@functools.partial(
    pl.kernel,
    out_type=(
        jax.ShapeDtypeStruct((NW, G1 * D), _f32),
        jax.ShapeDtypeStruct((NW, G1 * D), _f32),
        jax.ShapeDtypeStruct((NW, G1 * D), _f32),
    ),
    mesh=_mesh,
    scratch_types=[
        pltpu.VMEM((RPT * D,), _f32),
        pltpu.VMEM((RPT,), _i32),
        pltpu.VMEM((G1 * D,), _f32),
        pltpu.VMEM((G1 * D,), _f32),
        pltpu.VMEM((G1 * D,), _f32),
    ],
    compiler_params=pltpu.CompilerParams(needs_layout_passes=False),
)
def _pool_sc(h_hbm, batch_hbm, osum, omax, ocnt, rows_v, b_v, ls, lm, lc):
    c = lax.axis_index("c")
    s = lax.axis_index("s")
    w = c * NS + s
    pltpu.sync_copy(h_hbm.at[pl.ds(w * RPT * D, RPT * D)], rows_v)
    pltpu.sync_copy(batch_hbm.at[pl.ds(w * RPT, RPT)], b_v)

    def _zi(i, carry):
        ls[pl.ds(i * L, L)] = jnp.zeros((L,), _f32)
        lm[pl.ds(i * L, L)] = jnp.full((L,), -jnp.inf, _f32)
        lc[pl.ds(i * L, L)] = jnp.zeros((L,), _f32)
        return carry

    lax.fori_loop(0, (G1 * D) // L, _zi, 0)

    iota = lax.iota(_i32, L)
    ones = jnp.ones((L,), _f32)

    def _grp(g, carry):
        bvec = b_v[pl.ds(g * L, L)]
        for r in range(L):
            seg_b = jnp.take_along_axis(bvec, jnp.full((L,), r, _i32),
                                        axis=0, mode="promise_in_bounds")
            base = (g * L + r) * D
            sidx0 = seg_b * D + iota
            for j in range(D // L):
                rv = rows_v[pl.ds(base + j * L, L)]
                idx = sidx0 + (j * L)
                plsc.addupdate_scatter(ls, [idx], rv)
                curm = plsc.load_gather(lm, [idx])
                plsc.store_scatter(lm, [idx], jnp.maximum(curm, rv))
            plsc.addupdate_scatter(lc, [sidx0], ones)
        return carry

    lax.fori_loop(0, RPT // L, _grp, 0)
    pltpu.sync_copy(ls, osum.at[w])
    pltpu.sync_copy(lm, omax.at[w])
    pltpu.sync_copy(lc, ocnt.at[w])


def _stage0_body(degp, xp, w0, dinv_o, hn_o):
    deg = (degp[0, :] + degp[1, :] + 1.0).reshape(NP, 1)
    row = lax.broadcasted_iota(_i32, (NP, 1), 0)
    dinv = jnp.where(row < N, lax.rsqrt(deg), 0.0)
    dinv_o[...] = dinv
    hn = jnp.dot(xp[...], w0[...], preferred_element_type=_f32) * dinv
    hn_o[0] = hn[:, :HD]
    hn_o[1] = hn[:, HD:]


def _stage0(degp, xp, w0):
    return pl.pallas_call(
        _stage0_body,
        out_shape=(
            jax.ShapeDtypeStruct((NP, 1), _f32),
            jax.ShapeDtypeStruct((NC, NP, HD), _f32),
        ),
    )(degp, xp, w0)


def _stage_mid_body(accp, hn2, dinv, b, gm, be, wn, hn_o):
    row = lax.broadcasted_iota(_i32, (NP, 1), 0)
    acc = jnp.concatenate([accp[0], accp[1]], axis=1)
    hn = jnp.concatenate([hn2[0], hn2[1]], axis=1)
    pre = (acc + hn) * dinv[...] + b[...].reshape(1, D)
    pre = jnp.where(row < N, pre, 0.0)
    m = jnp.sum(pre, axis=0, keepdims=True) / N
    v = jnp.sum(pre * pre, axis=0, keepdims=True) / N - m * m
    y = (pre - m) * lax.rsqrt(v + 1e-5) * gm[...].reshape(1, D) + be[...].reshape(1, D)
    y = jnp.where(row < N, jnp.maximum(y, 0.0), 0.0)
    hn_n = jnp.dot(y, wn[...], preferred_element_type=_f32) * dinv[...]
    hn_o[0] = hn_n[:, :HD]
    hn_o[1] = hn_n[:, HD:]


def _stage_mid(accp, hn2, dinv, b, gm, be, wn):
    return pl.pallas_call(
        _stage_mid_body,
        out_shape=jax.ShapeDtypeStruct((NC, NP, HD), _f32),
    )(accp, hn2, dinv, b, gm, be, wn)


def _stage_last_body(accp, hn2, dinv, b, gm, be, h_o):
    row = lax.broadcasted_iota(_i32, (NP, 1), 0)
    acc = jnp.concatenate([accp[0], accp[1]], axis=1)
    hn = jnp.concatenate([hn2[0], hn2[1]], axis=1)
    pre = (acc + hn) * dinv[...] + b[...].reshape(1, D)
    pre = jnp.where(row < N, pre, 0.0)
    m = jnp.sum(pre, axis=0, keepdims=True) / N
    v = jnp.sum(pre * pre, axis=0, keepdims=True) / N - m * m
    y = (pre - m) * lax.rsqrt(v + 1e-5) * gm[...].reshape(1, D) + be[...].reshape(1, D)
    h_o[...] = jnp.where(row < N, jnp.maximum(y, 0.0), 0.0)


def _stage_last(accp, hn2, dinv, b, gm, be):
    return pl.pallas_call(
        _stage_last_body,
        out_shape=jax.ShapeDtypeStruct((NP, D), _f32),
    )(accp, hn2, dinv, b, gm, be)


def _head_body(psum, pmax, pcnt, fw1, fb1, fw2, fb2, fwo, fbo, out_o):
    ssum = jnp.sum(psum[...], axis=0).reshape(G1, D)[:G]
    smax = jnp.max(pmax[...], axis=0).reshape(G1, D)[:G]
    scnt = jnp.max(jnp.sum(pcnt[...], axis=0).reshape(G1, D),
                   axis=1, keepdims=True)[:G]
    mean = ssum / jnp.clip(scnt, 1.0)
    z = jnp.concatenate([mean, smax], axis=1)
    z = jnp.maximum(jnp.dot(z, fw1[...], preferred_element_type=_f32)
                    + fb1[...].reshape(1, D), 0.0)
    z = jnp.maximum(jnp.dot(z, fw2[...], preferred_element_type=_f32)
                    + fb2[...].reshape(1, D // 2), 0.0)
    out_o[...] = (jnp.dot(z, fwo[...], preferred_element_type=_f32)
                  + fbo[...].reshape(1, 5))


def _head(psum, pmax, pcnt, fw1, fb1, fw2, fb2, fwo, fbo):
    return pl.pallas_call(
        _head_body,
        out_shape=jax.ShapeDtypeStruct((G, 5), _f32),
    )(psum, pmax, pcnt, fw1, fb1, fw2, fb2, fwo, fbo)


def kernel(x, edge_index, batch, W0, b0, g0, be0, W1, b1, g1, be1,
           W2, b2, g2, be2, fW1, fb1, fW2, fb2, fWo, fbo):
    src = edge_index[0]
    dst = edge_index[1]
    npad = EP - E
    fill = jnp.arange(npad, dtype=_i32)
    src_p = jnp.concatenate([src, fill % N]).reshape(NS, KCH, CHUNK)
    dst_p = jnp.concatenate([dst, N + fill % (NP - N)]).reshape(NS, KCH, CHUNK)
    xp = jnp.pad(x, ((0, NP - N), (0, 0)))
    batch_p = jnp.concatenate([batch, jnp.full((NP - N,), G, _i32)])

    degp = _deg_sc(dst_p.reshape(NW, KD, CHUNK))
    dinv, hn2 = _stage0(degp, xp, W0)

    accp = _agg_sc(hn2, src_p, dst_p)
    hn2 = _stage_mid(accp, hn2, dinv, b0, g0, be0, W1)
    accp = _agg_sc(hn2, src_p, dst_p)
    hn2 = _stage_mid(accp, hn2, dinv, b1, g1, be1, W2)
    accp = _agg_sc(hn2, src_p, dst_p)
    h3 = _stage_last(accp, hn2, dinv, b2, g2, be2)

    psum, pmax, pcnt = _pool_sc(h3.reshape(NP * D), batch_p)
    return _head(psum, pmax, pcnt, fW1, fb1, fW2, fb2, fWo, fbo)


# speedup vs baseline: 25.4619x; 1.0366x over previous
import functools

import jax
import jax.numpy as jnp
from jax import lax
from jax.experimental import pallas as pl
from jax.experimental.pallas import tpu as pltpu
from jax.experimental.pallas import tpu_sc as plsc

N = 10000
E = 320000
D = 128
HD = D // 2
G = 64
G1 = G + 1
NC = 2
NS = 16
L = 16
NW = NC * NS
NP = 10240
RPT = NP // NW
CHUNK = 128
KCH = 158
EP = NS * KCH * CHUNK
KD = KCH // 2

_mesh = plsc.VectorSubcoreMesh(core_axis_name="c", subcore_axis_name="s")

_f32 = jnp.float32
_i32 = jnp.int32


@functools.partial(
    pl.kernel,
    out_type=jax.ShapeDtypeStruct((NC, NP), _f32),
    mesh=_mesh,
    scratch_types=[
        pltpu.VMEM((KD, CHUNK), _i32),
        pltpu.VMEM((CHUNK,), _f32),
        pltpu.VMEM((NP // NS,), _f32),
        pltpu.VMEM_SHARED((NP,), _f32),
    ] + [pltpu.SemaphoreType.DMA] * 4,
)
def _deg_sc(dsts_hbm, out_hbm, idx_v, ones_v, buf_v, acc_sh, *dsem):
    c = lax.axis_index("c")
    s = lax.axis_index("s")
    for k in range(CHUNK // L):
        ones_v[pl.ds(k * L, L)] = jnp.ones((L,), _f32)

    def _z(i, carry):
        buf_v[pl.ds(i * L, L)] = jnp.zeros((L,), _f32)
        return carry

    lax.fori_loop(0, (NP // NS) // L, _z, 0)
    pltpu.sync_copy(buf_v, acc_sh.at[pl.ds(s * (NP // NS), NP // NS)])
    w = c * NS + s
    pltpu.sync_copy(dsts_hbm.at[w], idx_v)
    plsc.subcore_barrier()
    dcp = [None] * 4
    for j in range(KD):
        b = j % 4
        if j >= 4:
            dcp[b].wait()
        dcp[b] = pltpu.async_copy(ones_v, acc_sh.at[idx_v.at[j]], dsem[b],
                                  add=True)
    for j in range(KD - 4, KD):
        dcp[j % 4].wait()
    plsc.subcore_barrier()
    pltpu.sync_copy(acc_sh.at[pl.ds(s * (NP // NS), NP // NS)], buf_v)
    pltpu.sync_copy(buf_v, out_hbm.at[c, pl.ds(s * (NP // NS), NP // NS)])


@functools.partial(
    pl.kernel,
    out_type=jax.ShapeDtypeStruct((NC, NP, HD), _f32),
    mesh=_mesh,
    scratch_types=[
        pltpu.VMEM((KCH, CHUNK), _i32),
        pltpu.VMEM((KCH, CHUNK), _i32),
        pltpu.VMEM((4, CHUNK, HD), _f32),
        pltpu.VMEM((2, CHUNK, HD), _f32),
        pltpu.VMEM_SHARED((NP, HD), _f32),
    ] + [pltpu.SemaphoreType.DMA] * 8,
    compiler_params=pltpu.CompilerParams(use_tc_tiling_on_sc=False),
)
def _agg_sc(hn_hbm, srcs_hbm, dsts_hbm, out_hbm,
            si_v, di_v, rows_v, wb_v, acc_sh, *sems):
    c = lax.axis_index("c")
    s = lax.axis_index("s")
    rows_per_tile = NP // NS
    NB = 4
    LAG = 2
    gsem = sems[:NB]
    ssem = sems[NB:2 * NB]
    zsem = gsem[0]
    wsa = (gsem[0], gsem[1])
    wsb = (gsem[2], gsem[3])

    def _z(r, carry):
        for j in range(HD // L):
            rows_v[0, r, pl.ds(j * L, L)] = jnp.zeros((L,), _f32)
        return carry

    lax.fori_loop(0, CHUNK, _z, 0)
    zcp = [None] * 5
    for k in range(rows_per_tile // CHUNK):
        zcp[k] = pltpu.async_copy(
            rows_v.at[0], acc_sh.at[pl.ds(s * rows_per_tile + k * CHUNK,
                                          CHUNK)], zsem)
    pltpu.sync_copy(srcs_hbm.at[s], si_v)
    pltpu.sync_copy(dsts_hbm.at[s], di_v)
    for k in range(rows_per_tile // CHUNK):
        zcp[k].wait()
    plsc.subcore_barrier()
    gcp = [None] * NB
    scp = [None] * NB
    for i in range(KCH + LAG):
        b = i % NB
        if i < KCH:
            if i >= NB:
                scp[b].wait()
            gcp[b] = pltpu.async_copy(hn_hbm.at[c].at[si_v.at[i]],
                                      rows_v.at[b], gsem[b])
        if i >= LAG:
            j = i - LAG
            bj = j % NB
            gcp[bj].wait()
            scp[bj] = pltpu.async_copy(rows_v.at[bj], acc_sh.at[di_v.at[j]],
                                       ssem[bj], add=True)
    for j in range(KCH - NB, KCH):
        scp[j % NB].wait()
    plsc.subcore_barrier()

    def _sl(k):
        return pl.ds(s * rows_per_tile + k * CHUNK, CHUNK)

    acp = [None, None]
    bcp = [None, None]
    acp[0] = pltpu.async_copy(acc_sh.at[_sl(0)], wb_v.at[0], wsa[0])
    for k in range(rows_per_tile // CHUNK):
        t = k % 2
        nt = (k + 1) % 2
        acp[t].wait()
        bcp[t] = pltpu.async_copy(wb_v.at[t], out_hbm.at[c, _sl(k)], wsb[t])
        if k + 1 < rows_per_tile // CHUNK:
            if k >= 1:
                bcp[nt].wait()
            acp[nt] = pltpu.async_copy(acc_sh.at[_sl(k + 1)], wb_v.at[nt],
                                       wsa[nt])
    bcp[(rows_per_tile // CHUNK - 2) % 2].wait()
    bcp[(rows_per_tile // CHUNK - 1) % 2].wait()


@functools.partial(
    pl.kernel,
    out_type=(
        jax.ShapeDtypeStruct((NW, G1 * D), _f32),
        jax.ShapeDtypeStruct((NW, G1 * D), _f32),
        jax.ShapeDtypeStruct((NW, G1 * D), _f32),
    ),
    mesh=_mesh,
    scratch_types=[
        pltpu.VMEM((RPT * D,), _f32),
        pltpu.VMEM((RPT,), _i32),
        pltpu.VMEM((G1 * D,), _f32),
        pltpu.VMEM((G1 * D,), _f32),
        pltpu.VMEM((G1 * D,), _f32),
    ],
    compiler_params=pltpu.CompilerParams(needs_layout_passes=False),
)
def _pool_sc(h_hbm, batch_hbm, osum, omax, ocnt, rows_v, b_v, ls, lm, lc):
    c = lax.axis_index("c")
    s = lax.axis_index("s")
    w = c * NS + s
    pltpu.sync_copy(h_hbm.at[pl.ds(w * RPT * D, RPT * D)], rows_v)
    pltpu.sync_copy(batch_hbm.at[pl.ds(w * RPT, RPT)], b_v)

    def _zi(i, carry):
        ls[pl.ds(i * L, L)] = jnp.zeros((L,), _f32)
        lm[pl.ds(i * L, L)] = jnp.full((L,), -jnp.inf, _f32)
        lc[pl.ds(i * L, L)] = jnp.zeros((L,), _f32)
        return carry

    lax.fori_loop(0, (G1 * D) // L, _zi, 0)

    iota = lax.iota(_i32, L)
    ones = jnp.ones((L,), _f32)

    def _grp(g, carry):
        bvec = b_v[pl.ds(g * L, L)]
        for r in range(L):
            seg_b = jnp.take_along_axis(bvec, jnp.full((L,), r, _i32),
                                        axis=0, mode="promise_in_bounds")
            base = (g * L + r) * D
            sidx0 = seg_b * D + iota
            for j in range(D // L):
                rv = rows_v[pl.ds(base + j * L, L)]
                idx = sidx0 + (j * L)
                plsc.addupdate_scatter(ls, [idx], rv)
                curm = plsc.load_gather(lm, [idx])
                plsc.store_scatter(lm, [idx], jnp.maximum(curm, rv))
            plsc.addupdate_scatter(lc, [sidx0], ones)
        return carry

    lax.fori_loop(0, RPT // L, _grp, 0)
    pltpu.sync_copy(ls, osum.at[w])
    pltpu.sync_copy(lm, omax.at[w])
    pltpu.sync_copy(lc, ocnt.at[w])


def _mm0_body(xp, w0, h_o):
    h_o[...] = jnp.dot(xp[...], w0[...], preferred_element_type=_f32)


def _mm0(xp, w0):
    return pl.pallas_call(
        _mm0_body,
        out_shape=jax.ShapeDtypeStruct((NP, D), _f32),
    )(xp, w0)


def _scale0_body(degp, h0, dinv_o, hn_o):
    deg = (degp[0, :] + degp[1, :] + 1.0).reshape(NP, 1)
    row = lax.broadcasted_iota(_i32, (NP, 1), 0)
    dinv = jnp.where(row < N, lax.rsqrt(deg), 0.0)
    dinv_o[...] = dinv
    hn = h0[...] * dinv
    hn_o[0] = hn[:, :HD]
    hn_o[1] = hn[:, HD:]


def _stage0(degp, h0):
    return pl.pallas_call(
        _scale0_body,
        out_shape=(
            jax.ShapeDtypeStruct((NP, 1), _f32),
            jax.ShapeDtypeStruct((NC, NP, HD), _f32),
        ),
    )(degp, h0)


def _stage_mid_body(accp, hn2, dinv, b, gm, be, wn, hn_o):
    row = lax.broadcasted_iota(_i32, (NP, 1), 0)
    acc = jnp.concatenate([accp[0], accp[1]], axis=1)
    hn = jnp.concatenate([hn2[0], hn2[1]], axis=1)
    pre = (acc + hn) * dinv[...] + b[...].reshape(1, D)
    pre = jnp.where(row < N, pre, 0.0)
    m = jnp.sum(pre, axis=0, keepdims=True) / N
    v = jnp.sum(pre * pre, axis=0, keepdims=True) / N - m * m
    y = (pre - m) * lax.rsqrt(v + 1e-5) * gm[...].reshape(1, D) + be[...].reshape(1, D)
    y = jnp.where(row < N, jnp.maximum(y, 0.0), 0.0)
    hn_n = jnp.dot(y, wn[...], preferred_element_type=_f32) * dinv[...]
    hn_o[0] = hn_n[:, :HD]
    hn_o[1] = hn_n[:, HD:]


def _stage_mid(accp, hn2, dinv, b, gm, be, wn):
    return pl.pallas_call(
        _stage_mid_body,
        out_shape=jax.ShapeDtypeStruct((NC, NP, HD), _f32),
    )(accp, hn2, dinv, b, gm, be, wn)


def _stage_last_body(accp, hn2, dinv, b, gm, be, h_o):
    row = lax.broadcasted_iota(_i32, (NP, 1), 0)
    acc = jnp.concatenate([accp[0], accp[1]], axis=1)
    hn = jnp.concatenate([hn2[0], hn2[1]], axis=1)
    pre = (acc + hn) * dinv[...] + b[...].reshape(1, D)
    pre = jnp.where(row < N, pre, 0.0)
    m = jnp.sum(pre, axis=0, keepdims=True) / N
    v = jnp.sum(pre * pre, axis=0, keepdims=True) / N - m * m
    y = (pre - m) * lax.rsqrt(v + 1e-5) * gm[...].reshape(1, D) + be[...].reshape(1, D)
    h_o[...] = jnp.where(row < N, jnp.maximum(y, 0.0), 0.0)


def _stage_last(accp, hn2, dinv, b, gm, be):
    return pl.pallas_call(
        _stage_last_body,
        out_shape=jax.ShapeDtypeStruct((NP, D), _f32),
    )(accp, hn2, dinv, b, gm, be)


def _head_body(psum, pmax, pcnt, fw1, fb1, fw2, fb2, fwo, fbo, out_o):
    ssum = jnp.sum(psum[...], axis=0).reshape(G1, D)[:G]
    smax = jnp.max(pmax[...], axis=0).reshape(G1, D)[:G]
    scnt = jnp.max(jnp.sum(pcnt[...], axis=0).reshape(G1, D),
                   axis=1, keepdims=True)[:G]
    mean = ssum / jnp.clip(scnt, 1.0)
    z = jnp.concatenate([mean, smax], axis=1)
    z = jnp.maximum(jnp.dot(z, fw1[...], preferred_element_type=_f32)
                    + fb1[...].reshape(1, D), 0.0)
    z = jnp.maximum(jnp.dot(z, fw2[...], preferred_element_type=_f32)
                    + fb2[...].reshape(1, D // 2), 0.0)
    out_o[...] = (jnp.dot(z, fwo[...], preferred_element_type=_f32)
                  + fbo[...].reshape(1, 5))


def _head(psum, pmax, pcnt, fw1, fb1, fw2, fb2, fwo, fbo):
    return pl.pallas_call(
        _head_body,
        out_shape=jax.ShapeDtypeStruct((G, 5), _f32),
    )(psum, pmax, pcnt, fw1, fb1, fw2, fb2, fwo, fbo)


def kernel(x, edge_index, batch, W0, b0, g0, be0, W1, b1, g1, be1,
           W2, b2, g2, be2, fW1, fb1, fW2, fb2, fWo, fbo):
    src = edge_index[0]
    dst = edge_index[1]
    npad = EP - E
    fill = jnp.arange(npad, dtype=_i32)
    src_p = jnp.concatenate([src, fill % N]).reshape(NS, KCH, CHUNK)
    dst_p = jnp.concatenate([dst, N + fill % (NP - N)]).reshape(NS, KCH, CHUNK)
    xp = jnp.pad(x, ((0, NP - N), (0, 0)))
    batch_p = jnp.concatenate([batch, jnp.full((NP - N,), G, _i32)])

    degp = _deg_sc(dst_p.reshape(NW, KD, CHUNK))
    h0 = _mm0(xp, W0)
    dinv, hn2 = _stage0(degp, h0)

    accp = _agg_sc(hn2, src_p, dst_p)
    hn2 = _stage_mid(accp, hn2, dinv, b0, g0, be0, W1)
    accp = _agg_sc(hn2, src_p, dst_p)
    hn2 = _stage_mid(accp, hn2, dinv, b1, g1, be1, W2)
    accp = _agg_sc(hn2, src_p, dst_p)
    h3 = _stage_last(accp, hn2, dinv, b2, g2, be2)

    psum, pmax, pcnt = _pool_sc(h3.reshape(NP * D), batch_p)
    return _head(psum, pmax, pcnt, fW1, fb1, fW2, fb2, fWo, fbo)


# speedup vs baseline: 26.2117x; 1.0294x over previous
import functools

import jax
import jax.numpy as jnp
from jax import lax
from jax.experimental import pallas as pl
from jax.experimental.pallas import tpu as pltpu
from jax.experimental.pallas import tpu_sc as plsc

N = 10000
E = 320000
D = 128
HD = D // 2
G = 64
G1 = G + 1
NC = 2
NS = 16
L = 16
NW = NC * NS
NP = 10240
RPT = NP // NW
CHUNK = 128
KCH = 162
EP = NS * KCH * CHUNK
KD = KCH // 2

_mesh = plsc.VectorSubcoreMesh(core_axis_name="c", subcore_axis_name="s")

_f32 = jnp.float32
_i32 = jnp.int32


@functools.partial(
    pl.kernel,
    out_type=jax.ShapeDtypeStruct((NC, NP), _f32),
    mesh=_mesh,
    scratch_types=[
        pltpu.VMEM((KD, CHUNK), _i32),
        pltpu.VMEM((CHUNK,), _f32),
        pltpu.VMEM((NP // NS,), _f32),
        pltpu.VMEM_SHARED((NP,), _f32),
    ] + [pltpu.SemaphoreType.DMA] * 4,
)
def _deg_sc(dsts_hbm, out_hbm, idx_v, ones_v, buf_v, acc_sh, *dsem):
    c = lax.axis_index("c")
    s = lax.axis_index("s")
    for k in range(CHUNK // L):
        ones_v[pl.ds(k * L, L)] = jnp.ones((L,), _f32)

    def _z(i, carry):
        buf_v[pl.ds(i * L, L)] = jnp.zeros((L,), _f32)
        return carry

    lax.fori_loop(0, (NP // NS) // L, _z, 0)
    pltpu.sync_copy(buf_v, acc_sh.at[pl.ds(s * (NP // NS), NP // NS)])
    w = c * NS + s
    pltpu.sync_copy(dsts_hbm.at[w], idx_v)
    plsc.subcore_barrier()
    dcp = [None] * 4
    for j in range(KD):
        b = j % 4
        if j >= 4:
            dcp[b].wait()
        dcp[b] = pltpu.async_copy(ones_v, acc_sh.at[idx_v.at[j]], dsem[b],
                                  add=True)
    for j in range(KD - 4, KD):
        dcp[j % 4].wait()
    plsc.subcore_barrier()
    pltpu.sync_copy(acc_sh.at[pl.ds(s * (NP // NS), NP // NS)], buf_v)
    pltpu.sync_copy(buf_v, out_hbm.at[c, pl.ds(s * (NP // NS), NP // NS)])


@functools.partial(
    pl.kernel,
    out_type=jax.ShapeDtypeStruct((NC, NP, HD), _f32),
    mesh=_mesh,
    scratch_types=[
        pltpu.VMEM((KCH, CHUNK), _i32),
        pltpu.VMEM((KCH, CHUNK), _i32),
        pltpu.VMEM((3, CHUNK, HD), _f32),
        pltpu.VMEM((2, CHUNK, HD), _f32),
        pltpu.VMEM_SHARED((NP, HD), _f32),
    ] + [pltpu.SemaphoreType.DMA] * 6,
    compiler_params=pltpu.CompilerParams(use_tc_tiling_on_sc=False),
)
def _agg_sc(hn_hbm, srcs_hbm, dsts_hbm, out_hbm,
            si_v, di_v, rows_v, wb_v, acc_sh, *sems):
    c = lax.axis_index("c")
    s = lax.axis_index("s")
    rows_per_tile = NP // NS
    NB = 3
    LAG = 2
    gsem = sems[:NB]
    ssem = sems[NB:2 * NB]
    zsem = gsem[0]
    wsa = (gsem[0], gsem[1])
    wsb = (gsem[2], ssem[0])

    def _z(r, carry):
        for j in range(HD // L):
            rows_v[0, r, pl.ds(j * L, L)] = jnp.zeros((L,), _f32)
        return carry

    lax.fori_loop(0, CHUNK, _z, 0)
    zcp = [None] * 5
    for k in range(rows_per_tile // CHUNK):
        zcp[k] = pltpu.async_copy(
            rows_v.at[0], acc_sh.at[pl.ds(s * rows_per_tile + k * CHUNK,
                                          CHUNK)], zsem)
    pltpu.sync_copy(srcs_hbm.at[s], si_v)
    pltpu.sync_copy(dsts_hbm.at[s], di_v)
    for k in range(rows_per_tile // CHUNK):
        zcp[k].wait()
    plsc.subcore_barrier()
    gcp = [None] * NB
    scp = [None] * NB
    for i in range(KCH + LAG):
        b = i % NB
        if i < KCH:
            if i >= NB:
                scp[b].wait()
            gcp[b] = pltpu.async_copy(hn_hbm.at[c].at[si_v.at[i]],
                                      rows_v.at[b], gsem[b])
        if i >= LAG:
            j = i - LAG
            bj = j % NB
            gcp[bj].wait()
            scp[bj] = pltpu.async_copy(rows_v.at[bj], acc_sh.at[di_v.at[j]],
                                       ssem[bj], add=True)
    for j in range(KCH - NB, KCH):
        scp[j % NB].wait()
    plsc.subcore_barrier()

    def _sl(k):
        return pl.ds(s * rows_per_tile + k * CHUNK, CHUNK)

    acp = [None, None]
    bcp = [None, None]
    acp[0] = pltpu.async_copy(acc_sh.at[_sl(0)], wb_v.at[0], wsa[0])
    for k in range(rows_per_tile // CHUNK):
        t = k % 2
        nt = (k + 1) % 2
        acp[t].wait()
        bcp[t] = pltpu.async_copy(wb_v.at[t], out_hbm.at[c, _sl(k)], wsb[t])
        if k + 1 < rows_per_tile // CHUNK:
            if k >= 1:
                bcp[nt].wait()
            acp[nt] = pltpu.async_copy(acc_sh.at[_sl(k + 1)], wb_v.at[nt],
                                       wsa[nt])
    bcp[(rows_per_tile // CHUNK - 2) % 2].wait()
    bcp[(rows_per_tile // CHUNK - 1) % 2].wait()


@functools.partial(
    pl.kernel,
    out_type=(
        jax.ShapeDtypeStruct((NW, G1 * D), _f32),
        jax.ShapeDtypeStruct((NW, G1 * D), _f32),
        jax.ShapeDtypeStruct((NW, G1 * D), _f32),
    ),
    mesh=_mesh,
    scratch_types=[
        pltpu.VMEM((RPT * D,), _f32),
        pltpu.VMEM((RPT,), _i32),
        pltpu.VMEM((G1 * D,), _f32),
        pltpu.VMEM((G1 * D,), _f32),
        pltpu.VMEM((G1 * D,), _f32),
    ],
    compiler_params=pltpu.CompilerParams(needs_layout_passes=False),
)
def _pool_sc(h_hbm, batch_hbm, osum, omax, ocnt, rows_v, b_v, ls, lm, lc):
    c = lax.axis_index("c")
    s = lax.axis_index("s")
    w = c * NS + s
    pltpu.sync_copy(h_hbm.at[pl.ds(w * RPT * D, RPT * D)], rows_v)
    pltpu.sync_copy(batch_hbm.at[pl.ds(w * RPT, RPT)], b_v)

    def _zi(i, carry):
        ls[pl.ds(i * L, L)] = jnp.zeros((L,), _f32)
        lm[pl.ds(i * L, L)] = jnp.full((L,), -jnp.inf, _f32)
        lc[pl.ds(i * L, L)] = jnp.zeros((L,), _f32)
        return carry

    lax.fori_loop(0, (G1 * D) // L, _zi, 0)

    iota = lax.iota(_i32, L)
    ones = jnp.ones((L,), _f32)

    def _grp(g, carry):
        bvec = b_v[pl.ds(g * L, L)]
        for r in range(L):
            seg_b = jnp.take_along_axis(bvec, jnp.full((L,), r, _i32),
                                        axis=0, mode="promise_in_bounds")
            base = (g * L + r) * D
            sidx0 = seg_b * D + iota
            for j in range(D // L):
                rv = rows_v[pl.ds(base + j * L, L)]
                idx = sidx0 + (j * L)
                plsc.addupdate_scatter(ls, [idx], rv)
                curm = plsc.load_gather(lm, [idx])
                plsc.store_scatter(lm, [idx], jnp.maximum(curm, rv))
            plsc.addupdate_scatter(lc, [sidx0], ones)
        return carry

    lax.fori_loop(0, RPT // L, _grp, 0)
    pltpu.sync_copy(ls, osum.at[w])
    pltpu.sync_copy(lm, omax.at[w])
    pltpu.sync_copy(lc, ocnt.at[w])


def _mm0_body(xp, w0, h_o):
    h_o[...] = jnp.dot(xp[...], w0[...], preferred_element_type=_f32)


def _mm0(xp, w0):
    return pl.pallas_call(
        _mm0_body,
        out_shape=jax.ShapeDtypeStruct((NP, D), _f32),
    )(xp, w0)


def _scale0_body(degp, h0, dinv_o, hn_o):
    deg = (degp[0, :] + degp[1, :]).reshape(NP, 1)
    row = lax.broadcasted_iota(_i32, (NP, 1), 0)
    dinv = jnp.where(row < N, lax.rsqrt(deg), 0.0)
    dinv_o[...] = dinv
    hn = h0[...] * dinv
    hn_o[0] = hn[:, :HD]
    hn_o[1] = hn[:, HD:]


def _stage0(degp, h0):
    return pl.pallas_call(
        _scale0_body,
        out_shape=(
            jax.ShapeDtypeStruct((NP, 1), _f32),
            jax.ShapeDtypeStruct((NC, NP, HD), _f32),
        ),
    )(degp, h0)


def _stage_mid_body(accp, dinv, b, gm, be, wn, hn_o):
    row = lax.broadcasted_iota(_i32, (NP, 1), 0)
    acc = jnp.concatenate([accp[0], accp[1]], axis=1)
    pre = acc * dinv[...] + b[...].reshape(1, D)
    pre = jnp.where(row < N, pre, 0.0)
    m = jnp.sum(pre, axis=0, keepdims=True) / N
    v = jnp.sum(pre * pre, axis=0, keepdims=True) / N - m * m
    y = (pre - m) * lax.rsqrt(v + 1e-5) * gm[...].reshape(1, D) + be[...].reshape(1, D)
    y = jnp.where(row < N, jnp.maximum(y, 0.0), 0.0)
    hn_n = jnp.dot(y, wn[...], preferred_element_type=_f32) * dinv[...]
    hn_o[0] = hn_n[:, :HD]
    hn_o[1] = hn_n[:, HD:]


def _stage_mid(accp, dinv, b, gm, be, wn):
    return pl.pallas_call(
        _stage_mid_body,
        out_shape=jax.ShapeDtypeStruct((NC, NP, HD), _f32),
    )(accp, dinv, b, gm, be, wn)


def _stage_last_body(accp, dinv, b, gm, be, h_o):
    row = lax.broadcasted_iota(_i32, (NP, 1), 0)
    acc = jnp.concatenate([accp[0], accp[1]], axis=1)
    pre = acc * dinv[...] + b[...].reshape(1, D)
    pre = jnp.where(row < N, pre, 0.0)
    m = jnp.sum(pre, axis=0, keepdims=True) / N
    v = jnp.sum(pre * pre, axis=0, keepdims=True) / N - m * m
    y = (pre - m) * lax.rsqrt(v + 1e-5) * gm[...].reshape(1, D) + be[...].reshape(1, D)
    h_o[...] = jnp.where(row < N, jnp.maximum(y, 0.0), 0.0)


def _stage_last(accp, dinv, b, gm, be):
    return pl.pallas_call(
        _stage_last_body,
        out_shape=jax.ShapeDtypeStruct((NP, D), _f32),
    )(accp, dinv, b, gm, be)


def _head_body(psum, pmax, pcnt, fw1, fb1, fw2, fb2, fwo, fbo, out_o):
    ssum = jnp.sum(psum[...], axis=0).reshape(G1, D)[:G]
    smax = jnp.max(pmax[...], axis=0).reshape(G1, D)[:G]
    scnt = jnp.max(jnp.sum(pcnt[...], axis=0).reshape(G1, D),
                   axis=1, keepdims=True)[:G]
    mean = ssum / jnp.clip(scnt, 1.0)
    z = jnp.concatenate([mean, smax], axis=1)
    z = jnp.maximum(jnp.dot(z, fw1[...], preferred_element_type=_f32)
                    + fb1[...].reshape(1, D), 0.0)
    z = jnp.maximum(jnp.dot(z, fw2[...], preferred_element_type=_f32)
                    + fb2[...].reshape(1, D // 2), 0.0)
    out_o[...] = (jnp.dot(z, fwo[...], preferred_element_type=_f32)
                  + fbo[...].reshape(1, 5))


def _head(psum, pmax, pcnt, fw1, fb1, fw2, fb2, fwo, fbo):
    return pl.pallas_call(
        _head_body,
        out_shape=jax.ShapeDtypeStruct((G, 5), _f32),
    )(psum, pmax, pcnt, fw1, fb1, fw2, fb2, fwo, fbo)


def kernel(x, edge_index, batch, W0, b0, g0, be0, W1, b1, g1, be1,
           W2, b2, g2, be2, fW1, fb1, fW2, fb2, fWo, fbo):
    src = edge_index[0]
    dst = edge_index[1]
    loop = jnp.arange(N, dtype=_i32)
    npad = EP - E - N
    fill = jnp.arange(npad, dtype=_i32)
    src_p = jnp.concatenate([src, loop, fill % N]).reshape(NS, KCH, CHUNK)
    dst_p = jnp.concatenate([dst, loop,
                             N + fill % (NP - N)]).reshape(NS, KCH, CHUNK)
    xp = jnp.pad(x, ((0, NP - N), (0, 0)))
    batch_p = jnp.concatenate([batch, jnp.full((NP - N,), G, _i32)])

    degp = _deg_sc(dst_p.reshape(NW, KD, CHUNK))
    h0 = _mm0(xp, W0)
    dinv, hn2 = _stage0(degp, h0)

    accp = _agg_sc(hn2, src_p, dst_p)
    hn2 = _stage_mid(accp, dinv, b0, g0, be0, W1)
    accp = _agg_sc(hn2, src_p, dst_p)
    hn2 = _stage_mid(accp, dinv, b1, g1, be1, W2)
    accp = _agg_sc(hn2, src_p, dst_p)
    h3 = _stage_last(accp, dinv, b2, g2, be2)

    psum, pmax, pcnt = _pool_sc(h3.reshape(NP * D), batch_p)
    return _head(psum, pmax, pcnt, fW1, fb1, fW2, fb2, fWo, fbo)


# speedup vs baseline: 26.4977x; 1.0109x over previous
import functools

import jax
import jax.numpy as jnp
from jax import lax
from jax.experimental import pallas as pl
from jax.experimental.pallas import tpu as pltpu
from jax.experimental.pallas import tpu_sc as plsc

N = 10000
E = 320000
D = 128
HD = D // 2
G = 64
G1 = G + 1
NC = 2
NS = 16
L = 16
NW = NC * NS
NP = 10240
RPT = NP // NW
CHUNK = 128
KCH = 162
EP = NS * KCH * CHUNK
KD = KCH // 2

_mesh = plsc.VectorSubcoreMesh(core_axis_name="c", subcore_axis_name="s")

_f32 = jnp.float32
_i32 = jnp.int32


@functools.partial(
    pl.kernel,
    out_type=jax.ShapeDtypeStruct((NC, NP), _f32),
    mesh=_mesh,
    scratch_types=[
        pltpu.VMEM((KD, CHUNK), _i32),
        pltpu.VMEM((CHUNK,), _f32),
        pltpu.VMEM((NP // NS,), _f32),
        pltpu.VMEM_SHARED((NP,), _f32),
    ] + [pltpu.SemaphoreType.DMA] * 4,
)
def _deg_sc(dsts_hbm, out_hbm, idx_v, ones_v, buf_v, acc_sh, *dsem):
    c = lax.axis_index("c")
    s = lax.axis_index("s")
    for k in range(CHUNK // L):
        ones_v[pl.ds(k * L, L)] = jnp.ones((L,), _f32)

    def _z(i, carry):
        buf_v[pl.ds(i * L, L)] = jnp.zeros((L,), _f32)
        return carry

    lax.fori_loop(0, (NP // NS) // L, _z, 0)
    pltpu.sync_copy(buf_v, acc_sh.at[pl.ds(s * (NP // NS), NP // NS)])
    w = c * NS + s
    pltpu.sync_copy(dsts_hbm.at[w], idx_v)
    plsc.subcore_barrier()
    dcp = [None] * 4
    for j in range(KD):
        b = j % 4
        if j >= 4:
            dcp[b].wait()
        dcp[b] = pltpu.async_copy(ones_v, acc_sh.at[idx_v.at[j]], dsem[b],
                                  add=True)
    for j in range(KD - 4, KD):
        dcp[j % 4].wait()
    plsc.subcore_barrier()
    pltpu.sync_copy(acc_sh.at[pl.ds(s * (NP // NS), NP // NS)], buf_v)
    pltpu.sync_copy(buf_v, out_hbm.at[c, pl.ds(s * (NP // NS), NP // NS)])


@functools.partial(
    pl.kernel,
    out_type=jax.ShapeDtypeStruct((NC, NP, HD), _f32),
    mesh=_mesh,
    scratch_types=[
        pltpu.VMEM((KCH, CHUNK), _i32),
        pltpu.VMEM((KCH, CHUNK), _i32),
        pltpu.VMEM((3, CHUNK, HD), _f32),
        pltpu.VMEM((2, CHUNK, HD), _f32),
        pltpu.VMEM_SHARED((NP, HD), _f32),
    ] + [pltpu.SemaphoreType.DMA] * 6,
    compiler_params=pltpu.CompilerParams(use_tc_tiling_on_sc=False),
)
def _agg_sc(hn_hbm, srcs_hbm, dsts_hbm, out_hbm,
            si_v, di_v, rows_v, wb_v, acc_sh, *sems):
    c = lax.axis_index("c")
    s = lax.axis_index("s")
    rows_per_tile = NP // NS
    NB = 3
    LAG = 2
    gsem = sems[:NB]
    ssem = sems[NB:2 * NB]
    zsem = gsem[0]
    wsa = (gsem[0], gsem[1])
    wsb = (gsem[2], ssem[0])

    def _z(r, carry):
        for j in range(HD // L):
            rows_v[0, r, pl.ds(j * L, L)] = jnp.zeros((L,), _f32)
        return carry

    lax.fori_loop(0, CHUNK, _z, 0)
    zcp = [None] * 5
    for k in range(rows_per_tile // CHUNK):
        zcp[k] = pltpu.async_copy(
            rows_v.at[0], acc_sh.at[pl.ds(s * rows_per_tile + k * CHUNK,
                                          CHUNK)], zsem)
    pltpu.sync_copy(srcs_hbm.at[s], si_v)
    pltpu.sync_copy(dsts_hbm.at[s], di_v)
    for k in range(rows_per_tile // CHUNK):
        zcp[k].wait()
    plsc.subcore_barrier()
    gcp = [None] * NB
    scp = [None] * NB
    for i in range(KCH + LAG):
        b = i % NB
        if i < KCH:
            if i >= NB:
                scp[b].wait()
            gcp[b] = pltpu.async_copy(hn_hbm.at[c].at[si_v.at[i]],
                                      rows_v.at[b], gsem[b])
        if i >= LAG:
            j = i - LAG
            bj = j % NB
            gcp[bj].wait()
            scp[bj] = pltpu.async_copy(rows_v.at[bj], acc_sh.at[di_v.at[j]],
                                       ssem[bj], add=True)
    for j in range(KCH - NB, KCH):
        scp[j % NB].wait()
    plsc.subcore_barrier()

    def _sl(k):
        return pl.ds(s * rows_per_tile + k * CHUNK, CHUNK)

    acp = [None, None]
    bcp = [None, None]
    acp[0] = pltpu.async_copy(acc_sh.at[_sl(0)], wb_v.at[0], wsa[0])
    for k in range(rows_per_tile // CHUNK):
        t = k % 2
        nt = (k + 1) % 2
        acp[t].wait()
        bcp[t] = pltpu.async_copy(wb_v.at[t], out_hbm.at[c, _sl(k)], wsb[t])
        if k + 1 < rows_per_tile // CHUNK:
            if k >= 1:
                bcp[nt].wait()
            acp[nt] = pltpu.async_copy(acc_sh.at[_sl(k + 1)], wb_v.at[nt],
                                       wsa[nt])
    bcp[(rows_per_tile // CHUNK - 2) % 2].wait()
    bcp[(rows_per_tile // CHUNK - 1) % 2].wait()


@functools.partial(
    pl.kernel,
    out_type=(
        jax.ShapeDtypeStruct((NW, G1 * D), _f32),
        jax.ShapeDtypeStruct((NW, G1 * D), _f32),
        jax.ShapeDtypeStruct((NW, G1 * D), _f32),
    ),
    mesh=_mesh,
    scratch_types=[
        pltpu.VMEM((RPT * D,), _f32),
        pltpu.VMEM((RPT,), _i32),
        pltpu.VMEM((G1 * D,), _f32),
        pltpu.VMEM((G1 * D,), _f32),
        pltpu.VMEM((G1 * D,), _f32),
        pltpu.VMEM((G1 * D,), _f32),
        pltpu.VMEM((G1 * D,), _f32),
        pltpu.VMEM((G1 * D,), _f32),
    ],
    compiler_params=pltpu.CompilerParams(needs_layout_passes=False),
)
def _pool_sc(h_hbm, batch_hbm, osum, omax, ocnt, rows_v, b_v, ls, lm, lc,
             ls2, lm2, lc2):
    c = lax.axis_index("c")
    s = lax.axis_index("s")
    w = c * NS + s
    pltpu.sync_copy(h_hbm.at[pl.ds(w * RPT * D, RPT * D)], rows_v)
    pltpu.sync_copy(batch_hbm.at[pl.ds(w * RPT, RPT)], b_v)

    def _zi(i, carry):
        ls[pl.ds(i * L, L)] = jnp.zeros((L,), _f32)
        lm[pl.ds(i * L, L)] = jnp.full((L,), -jnp.inf, _f32)
        lc[pl.ds(i * L, L)] = jnp.zeros((L,), _f32)
        ls2[pl.ds(i * L, L)] = jnp.zeros((L,), _f32)
        lm2[pl.ds(i * L, L)] = jnp.full((L,), -jnp.inf, _f32)
        lc2[pl.ds(i * L, L)] = jnp.zeros((L,), _f32)
        return carry

    lax.fori_loop(0, (G1 * D) // L, _zi, 0)

    iota = lax.iota(_i32, L)
    ones = jnp.ones((L,), _f32)
    half = RPT // 2

    def _grp(g, carry):
        bva = b_v[pl.ds(g * L, L)]
        bvb = b_v[pl.ds(half + g * L, L)]
        for r in range(L):
            rr = jnp.full((L,), r, _i32)
            sa = jnp.take_along_axis(bva, rr, axis=0,
                                     mode="promise_in_bounds")
            sb = jnp.take_along_axis(bvb, rr, axis=0,
                                     mode="promise_in_bounds")
            basea = (g * L + r) * D
            baseb = (half + g * L + r) * D
            ia0 = sa * D + iota
            ib0 = sb * D + iota
            for j in range(D // L):
                rva = rows_v[pl.ds(basea + j * L, L)]
                rvb = rows_v[pl.ds(baseb + j * L, L)]
                ia = ia0 + (j * L)
                ib = ib0 + (j * L)
                plsc.addupdate_scatter(ls, [ia], rva)
                plsc.addupdate_scatter(ls2, [ib], rvb)
                ca = plsc.load_gather(lm, [ia])
                cb = plsc.load_gather(lm2, [ib])
                plsc.store_scatter(lm, [ia], jnp.maximum(ca, rva))
                plsc.store_scatter(lm2, [ib], jnp.maximum(cb, rvb))
            plsc.addupdate_scatter(lc, [ia0], ones)
            plsc.addupdate_scatter(lc2, [ib0], ones)
        return carry

    lax.fori_loop(0, half // L, _grp, 0)

    def _mrg(i, carry):
        sl = pl.ds(i * L, L)
        ls[sl] = ls[sl] + ls2[sl]
        lm[sl] = jnp.maximum(lm[sl], lm2[sl])
        lc[sl] = lc[sl] + lc2[sl]
        return carry

    lax.fori_loop(0, (G1 * D) // L, _mrg, 0)
    pltpu.sync_copy(ls, osum.at[w])
    pltpu.sync_copy(lm, omax.at[w])
    pltpu.sync_copy(lc, ocnt.at[w])


def _mm0_body(xp, w0, h_o):
    h_o[...] = jnp.dot(xp[...], w0[...], preferred_element_type=_f32)


def _mm0(xp, w0):
    return pl.pallas_call(
        _mm0_body,
        out_shape=jax.ShapeDtypeStruct((NP, D), _f32),
    )(xp, w0)


def _scale0_body(degp, h0, dinv_o, hn_o):
    deg = (degp[0, :] + degp[1, :]).reshape(NP, 1)
    row = lax.broadcasted_iota(_i32, (NP, 1), 0)
    dinv = jnp.where(row < N, lax.rsqrt(deg), 0.0)
    dinv_o[...] = dinv
    hn = h0[...] * dinv
    hn_o[0] = hn[:, :HD]
    hn_o[1] = hn[:, HD:]


def _stage0(degp, h0):
    return pl.pallas_call(
        _scale0_body,
        out_shape=(
            jax.ShapeDtypeStruct((NP, 1), _f32),
            jax.ShapeDtypeStruct((NC, NP, HD), _f32),
        ),
    )(degp, h0)


def _stage_mid_body(accp, dinv, b, gm, be, wn, hn_o):
    row = lax.broadcasted_iota(_i32, (NP, 1), 0)
    acc = jnp.concatenate([accp[0], accp[1]], axis=1)
    pre = acc * dinv[...] + b[...].reshape(1, D)
    pre = jnp.where(row < N, pre, 0.0)
    m = jnp.sum(pre, axis=0, keepdims=True) / N
    v = jnp.sum(pre * pre, axis=0, keepdims=True) / N - m * m
    y = (pre - m) * lax.rsqrt(v + 1e-5) * gm[...].reshape(1, D) + be[...].reshape(1, D)
    y = jnp.where(row < N, jnp.maximum(y, 0.0), 0.0)
    hn_n = jnp.dot(y, wn[...], preferred_element_type=_f32) * dinv[...]
    hn_o[0] = hn_n[:, :HD]
    hn_o[1] = hn_n[:, HD:]


def _stage_mid(accp, dinv, b, gm, be, wn):
    return pl.pallas_call(
        _stage_mid_body,
        out_shape=jax.ShapeDtypeStruct((NC, NP, HD), _f32),
    )(accp, dinv, b, gm, be, wn)


def _stage_last_body(accp, dinv, b, gm, be, h_o):
    row = lax.broadcasted_iota(_i32, (NP, 1), 0)
    acc = jnp.concatenate([accp[0], accp[1]], axis=1)
    pre = acc * dinv[...] + b[...].reshape(1, D)
    pre = jnp.where(row < N, pre, 0.0)
    m = jnp.sum(pre, axis=0, keepdims=True) / N
    v = jnp.sum(pre * pre, axis=0, keepdims=True) / N - m * m
    y = (pre - m) * lax.rsqrt(v + 1e-5) * gm[...].reshape(1, D) + be[...].reshape(1, D)
    h_o[...] = jnp.where(row < N, jnp.maximum(y, 0.0), 0.0)


def _stage_last(accp, dinv, b, gm, be):
    return pl.pallas_call(
        _stage_last_body,
        out_shape=jax.ShapeDtypeStruct((NP, D), _f32),
    )(accp, dinv, b, gm, be)


def _head_body(psum, pmax, pcnt, fw1, fb1, fw2, fb2, fwo, fbo, out_o):
    ssum = jnp.sum(psum[...], axis=0).reshape(G1, D)[:G]
    smax = jnp.max(pmax[...], axis=0).reshape(G1, D)[:G]
    scnt = jnp.max(jnp.sum(pcnt[...], axis=0).reshape(G1, D),
                   axis=1, keepdims=True)[:G]
    mean = ssum / jnp.clip(scnt, 1.0)
    z = jnp.concatenate([mean, smax], axis=1)
    z = jnp.maximum(jnp.dot(z, fw1[...], preferred_element_type=_f32)
                    + fb1[...].reshape(1, D), 0.0)
    z = jnp.maximum(jnp.dot(z, fw2[...], preferred_element_type=_f32)
                    + fb2[...].reshape(1, D // 2), 0.0)
    out_o[...] = (jnp.dot(z, fwo[...], preferred_element_type=_f32)
                  + fbo[...].reshape(1, 5))


def _head(psum, pmax, pcnt, fw1, fb1, fw2, fb2, fwo, fbo):
    return pl.pallas_call(
        _head_body,
        out_shape=jax.ShapeDtypeStruct((G, 5), _f32),
    )(psum, pmax, pcnt, fw1, fb1, fw2, fb2, fwo, fbo)


def kernel(x, edge_index, batch, W0, b0, g0, be0, W1, b1, g1, be1,
           W2, b2, g2, be2, fW1, fb1, fW2, fb2, fWo, fbo):
    src = edge_index[0]
    dst = edge_index[1]
    loop = jnp.arange(N, dtype=_i32)
    npad = EP - E - N
    fill = jnp.arange(npad, dtype=_i32)
    src_p = jnp.concatenate([src, loop, fill % N]).reshape(NS, KCH, CHUNK)
    dst_p = jnp.concatenate([dst, loop,
                             N + fill % (NP - N)]).reshape(NS, KCH, CHUNK)
    xp = jnp.pad(x, ((0, NP - N), (0, 0)))
    batch_p = jnp.concatenate([batch, jnp.full((NP - N,), G, _i32)])

    degp = _deg_sc(dst_p.reshape(NW, KD, CHUNK))
    h0 = _mm0(xp, W0)
    dinv, hn2 = _stage0(degp, h0)

    accp = _agg_sc(hn2, src_p, dst_p)
    hn2 = _stage_mid(accp, dinv, b0, g0, be0, W1)
    accp = _agg_sc(hn2, src_p, dst_p)
    hn2 = _stage_mid(accp, dinv, b1, g1, be1, W2)
    accp = _agg_sc(hn2, src_p, dst_p)
    h3 = _stage_last(accp, dinv, b2, g2, be2)

    psum, pmax, pcnt = _pool_sc(h3.reshape(NP * D), batch_p)
    return _head(psum, pmax, pcnt, fW1, fb1, fW2, fb2, fWo, fbo)
